# baseline (device time: 88305 ns/iter reference)
import jax
import jax.numpy as jnp
from jax import lax
from jax.experimental import pallas as pl
from jax.experimental.pallas import tpu as pltpu

N_DEV = 4
E = 32
E_LOC = 8
CAP = 51
CAPP = 56
BLK = E_LOC * CAPP
HLF = BLK // 2
NSLOT = E * CAPP
N = 2048
D = 1024


def _fused_moe(x, route_idx, expert_W):
    def body(x_ref, ridx_ref, w_ref, out_ref,
             gbuf_ref, ybuf_ref, comm_ref, stok_ref, cnt_ref,
             send_sems, recv_sems):
        my = lax.axis_index("i")
        left = lax.rem(my + (N_DEV - 1), N_DEV)
        right = lax.rem(my + 1, N_DEV)

        barrier_sem = pltpu.get_barrier_semaphore()
        for nbr in (left, right):
            pl.semaphore_signal(
                barrier_sem, inc=1,
                device_id=(nbr,), device_id_type=pl.DeviceIdType.MESH,
            )

        def ibody(k, c):
            stok_ref[k] = N
            return c

        lax.fori_loop(0, NSLOT + 1, ibody, 0, unroll=8)

        def cbody(e, c):
            cnt_ref[e] = 0
            return c

        lax.fori_loop(0, E, cbody, 0, unroll=8)

        def abody(i, c):
            e = ridx_ref[i]
            cnt = cnt_ref[e]
            cnt_ref[e] = cnt + 1
            tgt = jnp.where(cnt < CAP, e * CAPP + cnt, NSLOT)
            stok_ref[tgt] = i
            return c

        lax.fori_loop(0, N, abody, 0)

        gbase = my * BLK

        def gbody(k, c):
            t = jnp.minimum(stok_ref[gbase + k], N - 1)
            gbuf_ref[pl.ds(k, 1), :] = x_ref[pl.ds(t, 1), :]
            return c

        lax.fori_loop(0, BLK, gbody, 0, unroll=8)

        for j in range(E_LOC):
            ybuf_ref[pl.ds(j * CAPP, CAPP), :] = jnp.dot(
                gbuf_ref[pl.ds(j * CAPP, CAPP), :],
                w_ref[j],
                preferred_element_type=jnp.float32,
            )
        comm_ref[0, :, :] = ybuf_ref[:, :].astype(jnp.bfloat16)

        def scatter_block(origin):
            sbase = origin * BLK

            def sbody(k, c):
                t = stok_ref[sbase + k]

                @pl.when(t < N)
                def _():
                    out_ref[pl.ds(t, 1), :] = ybuf_ref[pl.ds(k, 1), :]

                return c

            lax.fori_loop(0, BLK, sbody, 0, unroll=8)

        r1 = pltpu.make_async_remote_copy(
            src_ref=comm_ref.at[0], dst_ref=comm_ref.at[1],
            send_sem=send_sems.at[0], recv_sem=recv_sems.at[0],
            device_id=(right,), device_id_type=pl.DeviceIdType.MESH,
        )
        l1 = pltpu.make_async_remote_copy(
            src_ref=comm_ref.at[0], dst_ref=comm_ref.at[2],
            send_sem=send_sems.at[1], recv_sem=recv_sems.at[1],
            device_id=(left,), device_id_type=pl.DeviceIdType.MESH,
        )
        pl.semaphore_wait(barrier_sem, 2)
        r1.start()
        l1.start()

        out_ref[:, :] = jnp.zeros((N, D), jnp.float32)
        scatter_block(my)

        r1.wait_recv()
        r2 = pltpu.make_async_remote_copy(
            src_ref=comm_ref.at[1, pl.ds(0, HLF), :],
            dst_ref=comm_ref.at[3, pl.ds(0, HLF), :],
            send_sem=send_sems.at[2], recv_sem=recv_sems.at[2],
            device_id=(right,), device_id_type=pl.DeviceIdType.MESH,
        )
        r2.start()
        l1.wait_recv()
        l2 = pltpu.make_async_remote_copy(
            src_ref=comm_ref.at[2, pl.ds(HLF, HLF), :],
            dst_ref=comm_ref.at[3, pl.ds(HLF, HLF), :],
            send_sem=send_sems.at[3], recv_sem=recv_sems.at[3],
            device_id=(left,), device_id_type=pl.DeviceIdType.MESH,
        )
        l2.start()

        ybuf_ref[:, :] = comm_ref[1, :, :].astype(jnp.float32)
        scatter_block(left)
        ybuf_ref[:, :] = comm_ref[2, :, :].astype(jnp.float32)
        scatter_block(right)

        r2.wait_recv()
        l2.wait_recv()
        ybuf_ref[:, :] = comm_ref[3, :, :].astype(jnp.float32)
        scatter_block(lax.rem(my + 2, N_DEV))

        r1.wait_send()
        l1.wait_send()
        r2.wait_send()
        l2.wait_send()

    return pl.pallas_call(
        body,
        out_shape=jax.ShapeDtypeStruct((N, D), jnp.float32),
        in_specs=[
            pl.BlockSpec(memory_space=pltpu.VMEM),
            pl.BlockSpec(memory_space=pltpu.SMEM),
            pl.BlockSpec(memory_space=pltpu.VMEM),
        ],
        out_specs=pl.BlockSpec(memory_space=pltpu.VMEM),
        scratch_shapes=[
            pltpu.VMEM((BLK, D), jnp.float32),
            pltpu.VMEM((BLK, D), jnp.float32),
            pltpu.VMEM((N_DEV, BLK, D), jnp.bfloat16),
            pltpu.SMEM((NSLOT + 1,), jnp.int32),
            pltpu.SMEM((E,), jnp.int32),
            pltpu.SemaphoreType.DMA((4,)),
            pltpu.SemaphoreType.DMA((4,)),
        ],
        compiler_params=pltpu.CompilerParams(
            collective_id=0, vmem_limit_bytes=110 * 1024 * 1024
        ),
    )(x, route_idx, expert_W)


def kernel(x, router_W, route_idx, expert_W):
    del router_W
    return _fused_moe(x, route_idx.reshape(N).astype(jnp.int32), expert_W)
